# baseline (speedup 1.0000x reference)
import functools

import jax
import jax.numpy as jnp
from jax import lax
from jax.experimental import pallas as pl
from jax.experimental.pallas import tpu as pltpu
from jax.experimental.pallas import tpu_sc as plsc

BATCH = 16384
DIM = 64
NUM_CORES = 2
NUM_SUBCORES = 16
NUM_WORKERS = NUM_CORES * NUM_SUBCORES
B_PER_W = BATCH // NUM_WORKERS

_mesh = plsc.VectorSubcoreMesh(core_axis_name="c", subcore_axis_name="s")


@functools.partial(
    pl.kernel,
    mesh=_mesh,
    compiler_params=pltpu.CompilerParams(use_tc_tiling_on_sc=False),
    out_type=(
        jax.ShapeDtypeStruct((BATCH, DIM), jnp.float32),
        jax.ShapeDtypeStruct((BATCH,), jnp.int32),
    ),
    scratch_types=[
        pltpu.VMEM((B_PER_W,), jnp.int32),
        pltpu.VMEM((B_PER_W, DIM), jnp.float32),
        pltpu.VMEM((B_PER_W,), jnp.int32),
        pltpu.SemaphoreType.DMA,
        pltpu.SemaphoreType.DMA,
    ],
)
def _gather_kernel(nid_hbm, mem_hbm, last_hbm, mem_out_hbm, last_out_hbm,
                   idx_v, rows_v, last_v, sem_rows, sem_last):
    wid = lax.axis_index("s") * NUM_CORES + lax.axis_index("c")
    base = wid * B_PER_W
    pltpu.sync_copy(nid_hbm.at[pl.ds(base, B_PER_W)], idx_v)
    cp_rows = pltpu.async_copy(mem_hbm.at[idx_v], rows_v, sem_rows)
    cp_last = pltpu.async_copy(last_hbm.at[idx_v], last_v, sem_last)
    cp_rows.wait()
    pltpu.sync_copy(rows_v, mem_out_hbm.at[pl.ds(base, B_PER_W)])
    cp_last.wait()
    pltpu.sync_copy(last_v, last_out_hbm.at[pl.ds(base, B_PER_W)])


def kernel(n_id, memory, last_update):
    mem_out, last_out = _gather_kernel(
        n_id.astype(jnp.int32), memory, last_update)
    return (mem_out, last_out, jnp.array(0, dtype=jnp.int32))
